# baseline (speedup 1.0000x reference)
import jax
import jax.numpy as jnp
import numpy as np
from jax.experimental import pallas as pl
from jax.experimental.pallas import tpu as pltpu

N = 10000
F = 128
H1 = 32
H2 = 16

_BN = 200


def _decode_body(zb_ref, zfull_ref, out_ref):
    out_ref[...] = jax.lax.dot_general(
        zb_ref[...], zfull_ref[...],
        dimension_numbers=(((1,), (1,)), ((), ())),
        preferred_element_type=jnp.float32)


def _decode(z):
    return pl.pallas_call(
        _decode_body,
        grid=(N // _BN,),
        in_specs=[
            pl.BlockSpec((_BN, H2), lambda i: (i, 0)),
            pl.BlockSpec((N, H2), lambda i: (0, 0)),
        ],
        out_specs=pl.BlockSpec((_BN, N), lambda i: (i, 0)),
        out_shape=jax.ShapeDtypeStruct((N, N), jnp.float32),
    )(z, z)


def _gcn_layer(h, W, src, dst, ew):
    hw = h @ W
    msg = ew[:, None] * jnp.take(hw, src, axis=0)
    return jnp.zeros((N, W.shape[1]), dtype=hw.dtype).at[dst].add(msg)


def kernel(x, edge_index, edge_weight, W1, W2, W3):
    src = edge_index[1]
    dst = edge_index[0]
    h1 = jax.nn.relu(_gcn_layer(x, W1, src, dst, edge_weight))
    z_mean = _gcn_layer(h1, W2, src, dst, edge_weight)
    z_log_std = _gcn_layer(h1, W3, src, dst, edge_weight)
    eps = jax.random.normal(jax.random.key(42), (N, H2), dtype=jnp.float32)
    z = z_mean + eps * jnp.exp(z_log_std)
    return jnp.reshape(_decode(z), (-1,))


# speedup vs baseline: 4.2339x; 4.2339x over previous
import functools

import jax
import jax.numpy as jnp
from jax import lax
from jax.experimental import pallas as pl
from jax.experimental.pallas import tpu as pltpu
from jax.experimental.pallas import tpu_sc as plsc

N = 10000
E = 320000
F = 128
H1 = 32
H2 = 16

NC = 2
NS = 16
NW = NC * NS
CHUNK = 128
CPT = 79
EPT = CPT * CHUNK
E_PAD = NW * EPT
N_PAD = 10240
RPT = N_PAD // NS

_BN = 200

_mesh = plsc.VectorSubcoreMesh(core_axis_name="c", subcore_axis_name="s")


@functools.partial(
    pl.kernel,
    out_type=jax.ShapeDtypeStruct((NC, N_PAD, H1), jnp.float32),
    mesh=_mesh,
    scratch_types=[
        pltpu.VMEM_SHARED((N_PAD, H1), jnp.float32),
        pltpu.VMEM_SHARED((N_PAD, H1), jnp.float32),
        pltpu.VMEM((CPT, CHUNK), jnp.int32),
        pltpu.VMEM((CPT, CHUNK), jnp.int32),
        pltpu.VMEM((CHUNK, H1), jnp.float32),
        pltpu.VMEM((CHUNK, H1), jnp.float32),
    ],
)
def _sc_agg(hw_hbm, src_hbm, dst_hbm, ewb_hbm, zero_hbm, out_hbm,
            acc_sh, hw_sh, src_v, dst_v, rows_v, ewb_v):
    cid = lax.axis_index("c")
    sid = lax.axis_index("s")
    wid = sid * NC + cid
    r0 = sid * RPT
    pltpu.sync_copy(zero_hbm.at[pl.ds(r0, RPT)], acc_sh.at[pl.ds(r0, RPT)])
    pltpu.sync_copy(hw_hbm.at[pl.ds(r0, RPT)], hw_sh.at[pl.ds(r0, RPT)])
    plsc.subcore_barrier()
    pltpu.sync_copy(src_hbm.at[wid], src_v)
    pltpu.sync_copy(dst_hbm.at[wid], dst_v)
    ebase = wid * EPT

    @pl.loop(0, CPT)
    def _(j):
        pltpu.sync_copy(hw_sh.at[src_v.at[j]], rows_v)
        pltpu.sync_copy(ewb_hbm.at[pl.ds(ebase + j * CHUNK, CHUNK)], ewb_v)

        @pl.loop(0, CHUNK, unroll=4)
        def _(e):
            for c in range(0, H1, 16):
                rows_v[e, pl.ds(c, 16)] = (
                    rows_v[e, pl.ds(c, 16)] * ewb_v[e, pl.ds(c, 16)])

        pltpu.sync_copy(rows_v, acc_sh.at[dst_v.at[j]], add=True)

    plsc.subcore_barrier()
    pltpu.sync_copy(acc_sh.at[pl.ds(r0, RPT)],
                    out_hbm.at[cid, pl.ds(r0, RPT)])


def _mm1_body(x_ref, w_ref, o_ref):
    o_ref[...] = lax.dot_general(
        x_ref[...], w_ref[...], (((1,), (0,)), ((), ())),
        preferred_element_type=jnp.float32)


def _mm1(x, W1):
    return pl.pallas_call(
        _mm1_body,
        grid=(5,),
        in_specs=[pl.BlockSpec((N_PAD // 5, F), lambda i: (i, 0)),
                  pl.BlockSpec((F, H1), lambda i: (0, 0))],
        out_specs=pl.BlockSpec((N_PAD // 5, H1), lambda i: (i, 0)),
        out_shape=jax.ShapeDtypeStruct((N_PAD, H1), jnp.float32),
    )(x, W1)


def _ewb_body(e_ref, o_ref):
    o_ref[...] = jnp.broadcast_to(e_ref[...], (4096, H1))


def _ewb(ew_p2):
    return pl.pallas_call(
        _ewb_body,
        grid=(E_PAD // 4096,),
        in_specs=[pl.BlockSpec((4096, 1), lambda i: (i, 0))],
        out_specs=pl.BlockSpec((4096, H1), lambda i: (i, 0)),
        out_shape=jax.ShapeDtypeStruct((E_PAD, H1), jnp.float32),
    )(ew_p2)


def _comb2_body(p0_ref, p1_ref, w_ref, o_ref):
    h = jnp.maximum(p0_ref[...] + p1_ref[...], 0.0)
    o_ref[...] = lax.dot_general(
        h, w_ref[...], (((1,), (0,)), ((), ())),
        preferred_element_type=jnp.float32)


def _comb2(p0, p1, W23):
    return pl.pallas_call(
        _comb2_body,
        grid=(5,),
        in_specs=[pl.BlockSpec((N_PAD // 5, H1), lambda i: (i, 0)),
                  pl.BlockSpec((N_PAD // 5, H1), lambda i: (i, 0)),
                  pl.BlockSpec((H1, H1), lambda i: (0, 0))],
        out_specs=pl.BlockSpec((N_PAD // 5, H1), lambda i: (i, 0)),
        out_shape=jax.ShapeDtypeStruct((N_PAD, H1), jnp.float32),
    )(p0, p1, W23)


def _reparam_body(q0_ref, q1_ref, eps_ref, o_ref):
    s = q0_ref[...] + q1_ref[...]
    o_ref[...] = s[:, :H2] + eps_ref[...] * jnp.exp(s[:, H2:])


def _reparam(q0, q1, eps):
    return pl.pallas_call(
        _reparam_body,
        grid=(5,),
        in_specs=[pl.BlockSpec((2000, H1), lambda i: (i, 0)),
                  pl.BlockSpec((2000, H1), lambda i: (i, 0)),
                  pl.BlockSpec((2000, H2), lambda i: (i, 0))],
        out_specs=pl.BlockSpec((2000, H2), lambda i: (i, 0)),
        out_shape=jax.ShapeDtypeStruct((N, H2), jnp.float32),
    )(q0, q1, eps)


def _decode_body(zb_ref, zfull_ref, out_ref):
    out_ref[...] = lax.dot_general(
        zb_ref[...], zfull_ref[...], (((1,), (1,)), ((), ())),
        preferred_element_type=jnp.float32)


def _decode(z):
    return pl.pallas_call(
        _decode_body,
        grid=(N // _BN,),
        in_specs=[pl.BlockSpec((_BN, H2), lambda i: (i, 0)),
                  pl.BlockSpec((N, H2), lambda i: (0, 0))],
        out_specs=pl.BlockSpec((_BN, N), lambda i: (i, 0)),
        out_shape=jax.ShapeDtypeStruct((N, N), jnp.float32),
    )(z, z)


def kernel(x, edge_index, edge_weight, W1, W2, W3):
    src = edge_index[1]
    dst = edge_index[0]
    pad = E_PAD - E
    src_p = jnp.concatenate(
        [src, jnp.zeros((pad,), jnp.int32)]).reshape(NW, CPT, CHUNK)
    dst_p = jnp.concatenate(
        [dst, jnp.zeros((pad,), jnp.int32)]).reshape(NW, CPT, CHUNK)
    ew_p2 = jnp.concatenate(
        [edge_weight, jnp.zeros((pad,), jnp.float32)])[:, None]
    zeros_nh = jnp.zeros((N_PAD, H1), jnp.float32)
    W23 = jnp.concatenate([W2, W3], axis=1)
    eps = jax.random.normal(jax.random.key(42), (N, H2), dtype=jnp.float32)

    x_pad = jnp.concatenate(
        [x, jnp.zeros((N_PAD - N, F), jnp.float32)], axis=0)

    ewb = _ewb(ew_p2)
    hw1 = _mm1(x_pad, W1)
    p = _sc_agg(hw1, src_p, dst_p, ewb, zeros_nh)
    hw23 = _comb2(p[0], p[1], W23)
    q = _sc_agg(hw23, src_p, dst_p, ewb, zeros_nh)
    z = _reparam(q[0, :N], q[1, :N], eps)
    return jnp.reshape(_decode(z), (-1,))


# speedup vs baseline: 5.1993x; 1.2280x over previous
import functools

import jax
import jax.numpy as jnp
from jax import lax
from jax.experimental import pallas as pl
from jax.experimental.pallas import tpu as pltpu
from jax.experimental.pallas import tpu_sc as plsc

N = 10000
E = 320000
F = 128
H1 = 32
H2 = 16

NC = 2
NS = 16
NW = NC * NS
CHUNK = 128
SUP = 1
SUPE = SUP * CHUNK
NSUP = 80
CPT = NSUP * SUP
EPT = CPT * CHUNK
E_PAD = NW * EPT
N_PAD = 10240
RPT = N_PAD // NS

_BN = 200

_mesh = plsc.VectorSubcoreMesh(core_axis_name="c", subcore_axis_name="s")


@functools.partial(
    pl.kernel,
    out_type=jax.ShapeDtypeStruct((NC, N_PAD, H1), jnp.float32),
    mesh=_mesh,
    scratch_types=[
        pltpu.VMEM_SHARED((N_PAD, H1), jnp.float32),
        pltpu.VMEM_SHARED((N_PAD, H1), jnp.float32),
        pltpu.VMEM((CPT, CHUNK), jnp.int32),
        pltpu.VMEM((CPT, CHUNK), jnp.int32),
        pltpu.VMEM((2, SUPE, H1), jnp.float32),
        pltpu.VMEM((2, SUPE, H1), jnp.float32),
        pltpu.SemaphoreType.DMA,
        pltpu.SemaphoreType.DMA,
        pltpu.SemaphoreType.DMA,
        pltpu.SemaphoreType.DMA,
        pltpu.SemaphoreType.DMA,
        pltpu.SemaphoreType.DMA,
    ],
)
def _sc_agg(hw_hbm, src_hbm, dst_hbm, ewb_hbm, zero_hbm, out_hbm,
            acc_sh, hw_sh, src_v, dst_v, rows_v, ewb_v,
            gsem0, gsem1, esem0, esem1, ssem0, ssem1):
    gsem = (gsem0, gsem1)
    esem = (esem0, esem1)
    ssem = (ssem0, ssem1)
    cid = lax.axis_index("c")
    sid = lax.axis_index("s")
    wid = sid * NC + cid
    r0 = sid * RPT
    pltpu.sync_copy(zero_hbm.at[pl.ds(r0, RPT)], acc_sh.at[pl.ds(r0, RPT)])
    pltpu.sync_copy(hw_hbm.at[pl.ds(r0, RPT)], hw_sh.at[pl.ds(r0, RPT)])
    plsc.subcore_barrier()
    pltpu.sync_copy(src_hbm.at[wid], src_v)
    pltpu.sync_copy(dst_hbm.at[wid], dst_v)
    ebase = wid * EPT

    def fire_super(sb, b):
        for k in range(SUP):
            pltpu.async_copy(hw_sh.at[src_v.at[sb * SUP + k]],
                             rows_v.at[b, pl.ds(k * CHUNK, CHUNK)], gsem[b])
        pltpu.async_copy(ewb_hbm.at[pl.ds(ebase + sb * SUPE, SUPE)],
                         ewb_v.at[b], esem[b])

    def wait_super(sb, b):
        for k in range(SUP):
            pltpu.make_async_copy(
                hw_sh.at[src_v.at[sb * SUP + k]],
                rows_v.at[b, pl.ds(k * CHUNK, CHUNK)], gsem[b]).wait()
        pltpu.make_async_copy(ewb_hbm.at[pl.ds(ebase + sb * SUPE, SUPE)],
                              ewb_v.at[b], esem[b]).wait()

    def fire_scatter(sb, b):
        for k in range(SUP):
            pltpu.async_copy(rows_v.at[b, pl.ds(k * CHUNK, CHUNK)],
                             acc_sh.at[dst_v.at[sb * SUP + k]], ssem[b],
                             add=True)

    def wait_scatter(sb, b):
        for k in range(SUP):
            pltpu.make_async_copy(
                rows_v.at[b, pl.ds(k * CHUNK, CHUNK)],
                acc_sh.at[dst_v.at[sb * SUP + k]], ssem[b]).wait()

    fire_super(0, 0)

    @pl.loop(0, NSUP, step=2)
    def _(s):
        for ph in range(2):
            b = ph
            sb = s + ph

            @pl.when(sb > 0)
            def _():
                wait_scatter(sb - 1, 1 - b)

            @pl.when(sb < NSUP - 1)
            def _():
                fire_super(sb + 1, 1 - b)

            wait_super(sb, b)

            @pl.loop(0, SUPE, unroll=8)
            def _(e):
                for c in range(0, H1, 16):
                    rows_v[b, e, pl.ds(c, 16)] = (
                        rows_v[b, e, pl.ds(c, 16)]
                        * ewb_v[b, e, pl.ds(c, 16)])

            fire_scatter(sb, b)

    wait_scatter(NSUP - 1, (NSUP - 1) % 2)
    plsc.subcore_barrier()
    pltpu.sync_copy(acc_sh.at[pl.ds(r0, RPT)],
                    out_hbm.at[cid, pl.ds(r0, RPT)])


def _mm1_body(x_ref, w_ref, o_ref):
    o_ref[...] = lax.dot_general(
        x_ref[...], w_ref[...], (((1,), (0,)), ((), ())),
        preferred_element_type=jnp.float32)


def _mm1(x, W1):
    return pl.pallas_call(
        _mm1_body,
        grid=(5,),
        in_specs=[pl.BlockSpec((N_PAD // 5, F), lambda i: (i, 0)),
                  pl.BlockSpec((F, H1), lambda i: (0, 0))],
        out_specs=pl.BlockSpec((N_PAD // 5, H1), lambda i: (i, 0)),
        out_shape=jax.ShapeDtypeStruct((N_PAD, H1), jnp.float32),
    )(x, W1)


def _ewb_body(e_ref, o_ref):
    o_ref[...] = jnp.broadcast_to(e_ref[...], (4096, H1))


def _ewb(ew_p2):
    return pl.pallas_call(
        _ewb_body,
        grid=(E_PAD // 4096,),
        in_specs=[pl.BlockSpec((4096, 1), lambda i: (i, 0))],
        out_specs=pl.BlockSpec((4096, H1), lambda i: (i, 0)),
        out_shape=jax.ShapeDtypeStruct((E_PAD, H1), jnp.float32),
    )(ew_p2)


def _comb2_body(p0_ref, p1_ref, w_ref, o_ref):
    h = jnp.maximum(p0_ref[...] + p1_ref[...], 0.0)
    o_ref[...] = lax.dot_general(
        h, w_ref[...], (((1,), (0,)), ((), ())),
        preferred_element_type=jnp.float32)


def _comb2(p0, p1, W23):
    return pl.pallas_call(
        _comb2_body,
        grid=(5,),
        in_specs=[pl.BlockSpec((N_PAD // 5, H1), lambda i: (i, 0)),
                  pl.BlockSpec((N_PAD // 5, H1), lambda i: (i, 0)),
                  pl.BlockSpec((H1, H1), lambda i: (0, 0))],
        out_specs=pl.BlockSpec((N_PAD // 5, H1), lambda i: (i, 0)),
        out_shape=jax.ShapeDtypeStruct((N_PAD, H1), jnp.float32),
    )(p0, p1, W23)


def _reparam_body(q0_ref, q1_ref, eps_ref, o_ref):
    s = q0_ref[...] + q1_ref[...]
    o_ref[...] = s[:, :H2] + eps_ref[...] * jnp.exp(s[:, H2:])


def _reparam(q0, q1, eps):
    return pl.pallas_call(
        _reparam_body,
        grid=(5,),
        in_specs=[pl.BlockSpec((2000, H1), lambda i: (i, 0)),
                  pl.BlockSpec((2000, H1), lambda i: (i, 0)),
                  pl.BlockSpec((2000, H2), lambda i: (i, 0))],
        out_specs=pl.BlockSpec((2000, H2), lambda i: (i, 0)),
        out_shape=jax.ShapeDtypeStruct((N, H2), jnp.float32),
    )(q0, q1, eps)


def _decode_body(zb_ref, zfull_ref, out_ref):
    out_ref[...] = lax.dot_general(
        zb_ref[...], zfull_ref[...], (((1,), (1,)), ((), ())),
        preferred_element_type=jnp.float32)


def _decode(z):
    return pl.pallas_call(
        _decode_body,
        grid=(N // _BN,),
        in_specs=[pl.BlockSpec((_BN, H2), lambda i: (i, 0)),
                  pl.BlockSpec((N, H2), lambda i: (0, 0))],
        out_specs=pl.BlockSpec((_BN, N), lambda i: (i, 0)),
        out_shape=jax.ShapeDtypeStruct((N, N), jnp.float32),
    )(z, z)


def kernel(x, edge_index, edge_weight, W1, W2, W3):
    src = edge_index[1]
    dst = edge_index[0]
    pad = E_PAD - E
    src_p = jnp.concatenate(
        [src, jnp.zeros((pad,), jnp.int32)]).reshape(NW, CPT, CHUNK)
    dst_p = jnp.concatenate(
        [dst, jnp.zeros((pad,), jnp.int32)]).reshape(NW, CPT, CHUNK)
    ew_p2 = jnp.concatenate(
        [edge_weight, jnp.zeros((pad,), jnp.float32)])[:, None]
    zeros_nh = jnp.zeros((N_PAD, H1), jnp.float32)
    W23 = jnp.concatenate([W2, W3], axis=1)
    eps = jax.random.normal(jax.random.key(42), (N, H2), dtype=jnp.float32)

    x_pad = jnp.concatenate(
        [x, jnp.zeros((N_PAD - N, F), jnp.float32)], axis=0)

    ewb = _ewb(ew_p2)
    hw1 = _mm1(x_pad, W1)
    p = _sc_agg(hw1, src_p, dst_p, ewb, zeros_nh)
    hw23 = _comb2(p[0], p[1], W23)
    q = _sc_agg(hw23, src_p, dst_p, ewb, zeros_nh)
    z = _reparam(q[0, :N], q[1, :N], eps)
    return jnp.reshape(_decode(z), (-1,))


# speedup vs baseline: 7.5014x; 1.4428x over previous
import functools

import jax
import jax.numpy as jnp
import numpy as np
from jax import lax
from jax.experimental import pallas as pl
from jax.experimental.pallas import tpu as pltpu
from jax.experimental.pallas import tpu_sc as plsc

N = 10000
E = 320000
F = 128
H1 = 32
H2 = 16

NC = 2
NS = 16
NW = NC * NS
CHUNK = 128
SUP = 2
SUPE = SUP * CHUNK
NSUP = 40
CPT = NSUP * SUP
EPT = CPT * CHUNK
E_PAD = NW * EPT
N_PAD = 10240
RPT = N_PAD // NS

_BN = 200

_mesh = plsc.VectorSubcoreMesh(core_axis_name="c", subcore_axis_name="s")


@functools.partial(
    pl.kernel,
    out_type=jax.ShapeDtypeStruct((NC, N_PAD, H1), jnp.float32),
    mesh=_mesh,
    scratch_types=[
        pltpu.VMEM_SHARED((N_PAD, H1), jnp.float32),
        pltpu.VMEM_SHARED((N_PAD, H1), jnp.float32),
        pltpu.VMEM((CPT, CHUNK), jnp.int32),
        pltpu.VMEM((CPT, CHUNK), jnp.int32),
        pltpu.VMEM((2, SUPE, H1), jnp.float32),
        pltpu.VMEM((2, SUPE), jnp.float32),
        pltpu.SemaphoreType.DMA,
        pltpu.SemaphoreType.DMA,
        pltpu.SemaphoreType.DMA,
        pltpu.SemaphoreType.DMA,
        pltpu.SemaphoreType.DMA,
        pltpu.SemaphoreType.DMA,
    ],
)
def _sc_agg(hw_hbm, src_hbm, dst_hbm, ew_hbm, zero_hbm, out_hbm,
            acc_sh, hw_sh, src_v, dst_v, rows_v, ew_v,
            gsem0, gsem1, esem0, esem1, ssem0, ssem1):
    gsem = (gsem0, gsem1)
    esem = (esem0, esem1)
    ssem = (ssem0, ssem1)
    cid = lax.axis_index("c")
    sid = lax.axis_index("s")
    wid = sid * NC + cid
    r0 = sid * RPT
    pltpu.sync_copy(zero_hbm.at[pl.ds(r0, RPT)], acc_sh.at[pl.ds(r0, RPT)])
    pltpu.sync_copy(hw_hbm.at[pl.ds(r0, RPT)], hw_sh.at[pl.ds(r0, RPT)])
    plsc.subcore_barrier()
    pltpu.sync_copy(src_hbm.at[wid], src_v)
    pltpu.sync_copy(dst_hbm.at[wid], dst_v)
    ebase = wid * EPT

    def fire_super(sb, b):
        for k in range(SUP):
            pltpu.async_copy(hw_sh.at[src_v.at[sb * SUP + k]],
                             rows_v.at[b, pl.ds(k * CHUNK, CHUNK)], gsem[b])
        pltpu.async_copy(ew_hbm.at[pl.ds(ebase + sb * SUPE, SUPE)],
                         ew_v.at[b], esem[b])

    def wait_super(sb, b):
        for k in range(SUP):
            pltpu.make_async_copy(
                hw_sh.at[src_v.at[sb * SUP + k]],
                rows_v.at[b, pl.ds(k * CHUNK, CHUNK)], gsem[b]).wait()
        pltpu.make_async_copy(ew_hbm.at[pl.ds(ebase + sb * SUPE, SUPE)],
                              ew_v.at[b], esem[b]).wait()

    def fire_scatter(sb, b):
        for k in range(SUP):
            pltpu.async_copy(rows_v.at[b, pl.ds(k * CHUNK, CHUNK)],
                             acc_sh.at[dst_v.at[sb * SUP + k]], ssem[b],
                             add=True)

    def wait_scatter(sb, b):
        for k in range(SUP):
            pltpu.make_async_copy(
                rows_v.at[b, pl.ds(k * CHUNK, CHUNK)],
                acc_sh.at[dst_v.at[sb * SUP + k]], ssem[b]).wait()

    fire_super(0, 0)

    @pl.loop(0, NSUP, step=2)
    def _(s):
        for ph in range(2):
            b = ph
            sb = s + ph

            @pl.when(sb > 0)
            def _():
                wait_scatter(sb - 1, 1 - b)

            @pl.when(sb < NSUP - 1)
            def _():
                fire_super(sb + 1, 1 - b)

            wait_super(sb, b)

            @pl.loop(0, SUPE, step=16)
            def _(e0):
                wv = ew_v[b, pl.ds(e0, 16)]
                for k in range(16):
                    w = jnp.broadcast_to(wv[k:k + 1], (16,))
                    for c in range(0, H1, 16):
                        rows_v[b, e0 + k, pl.ds(c, 16)] = (
                            rows_v[b, e0 + k, pl.ds(c, 16)] * w)

            fire_scatter(sb, b)

    wait_scatter(NSUP - 1, (NSUP - 1) % 2)
    plsc.subcore_barrier()
    pltpu.sync_copy(acc_sh.at[pl.ds(r0, RPT)],
                    out_hbm.at[cid, pl.ds(r0, RPT)])


def _mm1_body(x_ref, w_ref, o_ref):
    o_ref[...] = lax.dot_general(
        x_ref[...], w_ref[...], (((1,), (0,)), ((), ())),
        preferred_element_type=jnp.float32)


def _mm1(x, W1):
    return pl.pallas_call(
        _mm1_body,
        grid=(5,),
        in_specs=[pl.BlockSpec((N_PAD // 5, F), lambda i: (i, 0)),
                  pl.BlockSpec((F, H1), lambda i: (0, 0))],
        out_specs=pl.BlockSpec((N_PAD // 5, H1), lambda i: (i, 0)),
        out_shape=jax.ShapeDtypeStruct((N_PAD, H1), jnp.float32),
    )(x, W1)


def _comb2_body(p0_ref, p1_ref, w_ref, o_ref):
    h = jnp.maximum(p0_ref[...] + p1_ref[...], 0.0)
    o_ref[...] = lax.dot_general(
        h, w_ref[...], (((1,), (0,)), ((), ())),
        preferred_element_type=jnp.float32)


def _comb2(p0, p1, W23):
    return pl.pallas_call(
        _comb2_body,
        grid=(5,),
        in_specs=[pl.BlockSpec((N_PAD // 5, H1), lambda i: (i, 0)),
                  pl.BlockSpec((N_PAD // 5, H1), lambda i: (i, 0)),
                  pl.BlockSpec((H1, H1), lambda i: (0, 0))],
        out_specs=pl.BlockSpec((N_PAD // 5, H1), lambda i: (i, 0)),
        out_shape=jax.ShapeDtypeStruct((N_PAD, H1), jnp.float32),
    )(p0, p1, W23)


def _reparam_body(q0_ref, q1_ref, eps_ref, o_ref):
    s = q0_ref[...] + q1_ref[...]
    o_ref[...] = s[:, :H2] + eps_ref[...] * jnp.exp(s[:, H2:])


def _reparam(q0, q1, eps):
    return pl.pallas_call(
        _reparam_body,
        grid=(5,),
        in_specs=[pl.BlockSpec((2000, H1), lambda i: (i, 0)),
                  pl.BlockSpec((2000, H1), lambda i: (i, 0)),
                  pl.BlockSpec((2000, H2), lambda i: (i, 0))],
        out_specs=pl.BlockSpec((2000, H2), lambda i: (i, 0)),
        out_shape=jax.ShapeDtypeStruct((N, H2), jnp.float32),
    )(q0, q1, eps)


def _decode_body(zb_ref, zfull_ref, out_ref):
    out_ref[...] = lax.dot_general(
        zb_ref[...], zfull_ref[...], (((1,), (1,)), ((), ())),
        preferred_element_type=jnp.float32)


def _decode(z):
    out = pl.pallas_call(
        _decode_body,
        grid=(N // _BN,),
        in_specs=[pl.BlockSpec((_BN, H2), lambda i: (i, 0)),
                  pl.BlockSpec((N, H2), lambda i: (0, 0))],
        out_specs=pl.BlockSpec((_BN, N), lambda i: (i, 0)),
        out_shape=jax.ShapeDtypeStruct((N, N), jnp.float32),
    )(z, z)
    return jnp.reshape(out, (N * N,))


def kernel(x, edge_index, edge_weight, W1, W2, W3):
    src = edge_index[1]
    dst = edge_index[0]
    pad = E_PAD - E
    src_p = jnp.concatenate(
        [src, jnp.zeros((pad,), jnp.int32)]).reshape(NW, CPT, CHUNK)
    dst_p = jnp.concatenate(
        [dst, jnp.zeros((pad,), jnp.int32)]).reshape(NW, CPT, CHUNK)
    ew_p = jnp.concatenate(
        [edge_weight, jnp.zeros((pad,), jnp.float32)])
    zeros_nh = jnp.zeros((N_PAD, H1), jnp.float32)
    W23 = jnp.concatenate([W2, W3], axis=1)
    eps = jax.random.normal(jax.random.key(42), (N, H2), dtype=jnp.float32)

    x_pad = jnp.concatenate(
        [x, jnp.zeros((N_PAD - N, F), jnp.float32)], axis=0)

    hw1 = _mm1(x_pad, W1)
    p = _sc_agg(hw1, src_p, dst_p, ew_p, zeros_nh)
    hw23 = _comb2(p[0], p[1], W23)
    q = _sc_agg(hw23, src_p, dst_p, ew_p, zeros_nh)
    z = _reparam(q[0, :N], q[1, :N], eps)
    return _decode(z)


# speedup vs baseline: 11.3227x; 1.5094x over previous
import functools

import jax
import jax.numpy as jnp
import numpy as np
from jax import lax
from jax.experimental import pallas as pl
from jax.experimental.pallas import tpu as pltpu
from jax.experimental.pallas import tpu_sc as plsc

N = 10000
E = 320000
F = 128
H1 = 32
H2 = 16

NC = 2
NS = 16
NW = NC * NS
CHUNK = 128
SUP = 2
SUPE = SUP * CHUNK
NSUP = 40
CPT = NSUP * SUP
EPT = CPT * CHUNK
E_PAD = NW * EPT
N_PAD = 10240
RPT = N_PAD // NS

_BN = 200

_mesh = plsc.VectorSubcoreMesh(core_axis_name="c", subcore_axis_name="s")


@functools.partial(
    pl.kernel,
    out_type=jax.ShapeDtypeStruct((NC, N_PAD, H1), jnp.float32),
    mesh=_mesh,
    scratch_types=[
        pltpu.VMEM_SHARED((N_PAD, H1), jnp.float32),
        pltpu.VMEM_SHARED((N_PAD, H1), jnp.float32),
        pltpu.VMEM((CPT, CHUNK), jnp.int32),
        pltpu.VMEM((CPT, CHUNK), jnp.int32),
        pltpu.VMEM((2, SUPE, H1), jnp.float32),
        pltpu.VMEM((2, SUPE), jnp.float32),
        pltpu.SemaphoreType.DMA,
        pltpu.SemaphoreType.DMA,
        pltpu.SemaphoreType.DMA,
        pltpu.SemaphoreType.DMA,
        pltpu.SemaphoreType.DMA,
        pltpu.SemaphoreType.DMA,
    ],
)
def _sc_agg(hw_hbm, src_hbm, dst_hbm, ew_hbm, zero_hbm, out_hbm,
            acc_sh, hw_sh, src_v, dst_v, rows_v, ew_v,
            gsem0, gsem1, esem0, esem1, ssem0, ssem1):
    gsem = (gsem0, gsem1)
    esem = (esem0, esem1)
    ssem = (ssem0, ssem1)
    cid = lax.axis_index("c")
    sid = lax.axis_index("s")
    wid = sid * NC + cid
    r0 = sid * RPT
    pltpu.sync_copy(zero_hbm.at[pl.ds(r0, RPT)], acc_sh.at[pl.ds(r0, RPT)])
    pltpu.sync_copy(hw_hbm.at[pl.ds(r0, RPT)], hw_sh.at[pl.ds(r0, RPT)])
    plsc.subcore_barrier()
    pltpu.sync_copy(src_hbm.at[wid], src_v)
    pltpu.sync_copy(dst_hbm.at[wid], dst_v)
    ebase = wid * EPT

    def fire_super(sb, b):
        for k in range(SUP):
            pltpu.async_copy(hw_sh.at[src_v.at[sb * SUP + k]],
                             rows_v.at[b, pl.ds(k * CHUNK, CHUNK)], gsem[b])
        pltpu.async_copy(ew_hbm.at[pl.ds(ebase + sb * SUPE, SUPE)],
                         ew_v.at[b], esem[b])

    def wait_super(sb, b):
        for k in range(SUP):
            pltpu.make_async_copy(
                hw_sh.at[src_v.at[sb * SUP + k]],
                rows_v.at[b, pl.ds(k * CHUNK, CHUNK)], gsem[b]).wait()
        pltpu.make_async_copy(ew_hbm.at[pl.ds(ebase + sb * SUPE, SUPE)],
                              ew_v.at[b], esem[b]).wait()

    def fire_scatter(sb, b):
        for k in range(SUP):
            pltpu.async_copy(rows_v.at[b, pl.ds(k * CHUNK, CHUNK)],
                             acc_sh.at[dst_v.at[sb * SUP + k]], ssem[b],
                             add=True)

    def wait_scatter(sb, b):
        for k in range(SUP):
            pltpu.make_async_copy(
                rows_v.at[b, pl.ds(k * CHUNK, CHUNK)],
                acc_sh.at[dst_v.at[sb * SUP + k]], ssem[b]).wait()

    fire_super(0, 0)

    @pl.loop(0, NSUP, step=2)
    def _(s):
        for ph in range(2):
            b = ph
            sb = s + ph

            @pl.when(sb > 0)
            def _():
                wait_scatter(sb - 1, 1 - b)

            @pl.when(sb < NSUP - 1)
            def _():
                fire_super(sb + 1, 1 - b)

            wait_super(sb, b)

            @pl.loop(0, SUPE, step=16)
            def _(e0):
                wv = ew_v[b, pl.ds(e0, 16)]
                for k in range(16):
                    w = jnp.broadcast_to(wv[k:k + 1], (16,))
                    for c in range(0, H1, 16):
                        rows_v[b, e0 + k, pl.ds(c, 16)] = (
                            rows_v[b, e0 + k, pl.ds(c, 16)] * w)

            fire_scatter(sb, b)

    wait_scatter(NSUP - 1, (NSUP - 1) % 2)
    plsc.subcore_barrier()
    pltpu.sync_copy(acc_sh.at[pl.ds(r0, RPT)],
                    out_hbm.at[cid, pl.ds(r0, RPT)])


def _mm1_body(x_ref, w_ref, o_ref):
    o_ref[...] = lax.dot_general(
        x_ref[...], w_ref[...], (((1,), (0,)), ((), ())),
        preferred_element_type=jnp.float32)


def _mm1(x, W1):
    return pl.pallas_call(
        _mm1_body,
        grid=(5,),
        in_specs=[pl.BlockSpec((N_PAD // 5, F), lambda i: (i, 0)),
                  pl.BlockSpec((F, H1), lambda i: (0, 0))],
        out_specs=pl.BlockSpec((N_PAD // 5, H1), lambda i: (i, 0)),
        out_shape=jax.ShapeDtypeStruct((N_PAD, H1), jnp.float32),
    )(x, W1)


def _comb2_body(p0_ref, p1_ref, w_ref, o_ref):
    h = jnp.maximum(p0_ref[...] + p1_ref[...], 0.0)
    o_ref[...] = lax.dot_general(
        h, w_ref[...], (((1,), (0,)), ((), ())),
        preferred_element_type=jnp.float32)


def _comb2(p0, p1, W23):
    return pl.pallas_call(
        _comb2_body,
        grid=(5,),
        in_specs=[pl.BlockSpec((N_PAD // 5, H1), lambda i: (i, 0)),
                  pl.BlockSpec((N_PAD // 5, H1), lambda i: (i, 0)),
                  pl.BlockSpec((H1, H1), lambda i: (0, 0))],
        out_specs=pl.BlockSpec((N_PAD // 5, H1), lambda i: (i, 0)),
        out_shape=jax.ShapeDtypeStruct((N_PAD, H1), jnp.float32),
    )(p0, p1, W23)


def _reparam_body(q0_ref, q1_ref, eps_ref, o_ref):
    s = q0_ref[...] + q1_ref[...]
    o_ref[...] = s[:, :H2] + eps_ref[...] * jnp.exp(s[:, H2:])


def _reparam(q0, q1, eps):
    return pl.pallas_call(
        _reparam_body,
        grid=(5,),
        in_specs=[pl.BlockSpec((2000, H1), lambda i: (i, 0)),
                  pl.BlockSpec((2000, H1), lambda i: (i, 0)),
                  pl.BlockSpec((2000, H2), lambda i: (i, 0))],
        out_specs=pl.BlockSpec((2000, H2), lambda i: (i, 0)),
        out_shape=jax.ShapeDtypeStruct((N, H2), jnp.float32),
    )(q0, q1, eps)


_ZB = 64
_GB = (N + _ZB - 1) // _ZB
_NP = _GB * _ZB
_OBR = _ZB * N // 128


def _decode_body(zb_ref, ztr_ref, out_ref):
    lane = lax.broadcasted_iota(jnp.int32, (8, 128), 1)
    d = [lax.dot_general(zb_ref[8 * s:8 * s + 8, :], ztr_ref[s],
                         (((1,), (0,)), ((), ())),
                         preferred_element_type=jnp.float32)
         for s in range(8)]
    for s in range(8):
        main = jnp.reshape(d[s][:, :9984], (624, 128))
        for m in range(8):
            out_ref[pl.ds(625 * m + 78 * s, 78), :] = main[78 * m:78 * m + 78, :]
    for s in range(8):
        tail = d[s][:, 9984:10112]
        if s < 7:
            bnd = jnp.where(lane < 16 * (s + 1), tail, d[s + 1][:, 0:128])
        else:
            bnd = tail
        for m in range(8):
            out_ref[pl.ds(625 * m + 78 * s + 78, 1), :] = bnd[m:m + 1, :]


def _decode(z):
    zp = jnp.concatenate([z, jnp.zeros((_NP - N, H2), jnp.float32)], axis=0)
    zperm = jnp.reshape(
        jnp.transpose(jnp.reshape(zp, (_GB, 8, 8, H2)), (0, 2, 1, 3)),
        (_NP, H2))
    zT = z.T
    rolls = jnp.stack([jnp.roll(zT, 16 * s, axis=1) for s in range(8)])
    ztr = jnp.concatenate([rolls, rolls[:, :, :112]], axis=2)
    out2 = pl.pallas_call(
        _decode_body,
        grid=(_GB,),
        in_specs=[pl.BlockSpec((_ZB, H2), lambda g: (g, 0)),
                  pl.BlockSpec((8, H2, N + 112), lambda g: (0, 0, 0))],
        out_specs=pl.BlockSpec((_OBR, 128), lambda g: (g, 0)),
        out_shape=jax.ShapeDtypeStruct((N * N // 128, 128), jnp.float32),
    )(zperm, ztr)
    return jnp.reshape(out2, (N * N,))


def kernel(x, edge_index, edge_weight, W1, W2, W3):
    src = edge_index[1]
    dst = edge_index[0]
    pad = E_PAD - E
    src_p = jnp.concatenate(
        [src, jnp.zeros((pad,), jnp.int32)]).reshape(NW, CPT, CHUNK)
    dst_p = jnp.concatenate(
        [dst, jnp.zeros((pad,), jnp.int32)]).reshape(NW, CPT, CHUNK)
    ew_p = jnp.concatenate(
        [edge_weight, jnp.zeros((pad,), jnp.float32)])
    zeros_nh = jnp.zeros((N_PAD, H1), jnp.float32)
    W23 = jnp.concatenate([W2, W3], axis=1)
    eps = jax.random.normal(jax.random.key(42), (N, H2), dtype=jnp.float32)

    x_pad = jnp.concatenate(
        [x, jnp.zeros((N_PAD - N, F), jnp.float32)], axis=0)

    hw1 = _mm1(x_pad, W1)
    p = _sc_agg(hw1, src_p, dst_p, ew_p, zeros_nh)
    hw23 = _comb2(p[0], p[1], W23)
    q = _sc_agg(hw23, src_p, dst_p, ew_p, zeros_nh)
    z = _reparam(q[0, :N], q[1, :N], eps)
    return _decode(z)


# speedup vs baseline: 11.7383x; 1.0367x over previous
import functools

import jax
import jax.numpy as jnp
import numpy as np
from jax import lax
from jax.experimental import pallas as pl
from jax.experimental.pallas import tpu as pltpu
from jax.experimental.pallas import tpu_sc as plsc

N = 10000
E = 320000
F = 128
H1 = 32
H2 = 16

NC = 2
NS = 16
NW = NC * NS
CHUNK = 128
SUP = 2
SUPE = SUP * CHUNK
NSUP = 40
CPT = NSUP * SUP
EPT = CPT * CHUNK
E_PAD = NW * EPT
N_PAD = 10240
RPT = N_PAD // NS

_BN = 200

_mesh = plsc.VectorSubcoreMesh(core_axis_name="c", subcore_axis_name="s")


@functools.partial(
    pl.kernel,
    out_type=jax.ShapeDtypeStruct((NC, N_PAD, H1), jnp.float32),
    mesh=_mesh,
    scratch_types=[
        pltpu.VMEM_SHARED((N_PAD, H1), jnp.float32),
        pltpu.VMEM_SHARED((N_PAD, H1), jnp.float32),
        pltpu.VMEM((CPT, CHUNK), jnp.int32),
        pltpu.VMEM((CPT, CHUNK), jnp.int32),
        pltpu.VMEM((2, SUPE, H1), jnp.float32),
        pltpu.VMEM((2, SUPE), jnp.float32),
        pltpu.SemaphoreType.DMA,
        pltpu.SemaphoreType.DMA,
        pltpu.SemaphoreType.DMA,
        pltpu.SemaphoreType.DMA,
        pltpu.SemaphoreType.DMA,
        pltpu.SemaphoreType.DMA,
    ],
)
def _sc_agg(hw_hbm, src_hbm, dst_hbm, ew_hbm, zero_hbm, out_hbm,
            acc_sh, hw_sh, src_v, dst_v, rows_v, ew_v,
            gsem0, gsem1, esem0, esem1, ssem0, ssem1):
    gsem = (gsem0, gsem1)
    esem = (esem0, esem1)
    ssem = (ssem0, ssem1)
    cid = lax.axis_index("c")
    sid = lax.axis_index("s")
    wid = sid * NC + cid
    r0 = sid * RPT
    pltpu.sync_copy(zero_hbm.at[pl.ds(r0, RPT)], acc_sh.at[pl.ds(r0, RPT)])
    pltpu.sync_copy(hw_hbm.at[pl.ds(r0, RPT)], hw_sh.at[pl.ds(r0, RPT)])
    plsc.subcore_barrier()
    pltpu.sync_copy(src_hbm.at[wid], src_v)
    pltpu.sync_copy(dst_hbm.at[wid], dst_v)
    ebase = wid * EPT

    def fire_super(sb, b):
        for k in range(SUP):
            pltpu.async_copy(hw_sh.at[src_v.at[sb * SUP + k]],
                             rows_v.at[b, pl.ds(k * CHUNK, CHUNK)], gsem[b])
        pltpu.async_copy(ew_hbm.at[pl.ds(ebase + sb * SUPE, SUPE)],
                         ew_v.at[b], esem[b])

    def wait_super(sb, b):
        for k in range(SUP):
            pltpu.make_async_copy(
                hw_sh.at[src_v.at[sb * SUP + k]],
                rows_v.at[b, pl.ds(k * CHUNK, CHUNK)], gsem[b]).wait()
        pltpu.make_async_copy(ew_hbm.at[pl.ds(ebase + sb * SUPE, SUPE)],
                              ew_v.at[b], esem[b]).wait()

    def fire_scatter(sb, b):
        for k in range(SUP):
            pltpu.async_copy(rows_v.at[b, pl.ds(k * CHUNK, CHUNK)],
                             acc_sh.at[dst_v.at[sb * SUP + k]], ssem[b],
                             add=True)

    def wait_scatter(sb, b):
        for k in range(SUP):
            pltpu.make_async_copy(
                rows_v.at[b, pl.ds(k * CHUNK, CHUNK)],
                acc_sh.at[dst_v.at[sb * SUP + k]], ssem[b]).wait()

    fire_super(0, 0)

    @pl.loop(0, NSUP, step=2)
    def _(s):
        for ph in range(2):
            b = ph
            sb = s + ph

            @pl.when(sb > 0)
            def _():
                wait_scatter(sb - 1, 1 - b)

            @pl.when(sb < NSUP - 1)
            def _():
                fire_super(sb + 1, 1 - b)

            wait_super(sb, b)

            @pl.loop(0, SUPE, step=16)
            def _(e0):
                wv = ew_v[b, pl.ds(e0, 16)]
                for k in range(16):
                    w = jnp.broadcast_to(wv[k:k + 1], (16,))
                    for c in range(0, H1, 16):
                        rows_v[b, e0 + k, pl.ds(c, 16)] = (
                            rows_v[b, e0 + k, pl.ds(c, 16)] * w)

            fire_scatter(sb, b)

    wait_scatter(NSUP - 1, (NSUP - 1) % 2)
    plsc.subcore_barrier()
    pltpu.sync_copy(acc_sh.at[pl.ds(r0, RPT)],
                    out_hbm.at[cid, pl.ds(r0, RPT)])


def _mm1_body(x_ref, w_ref, o_ref):
    o_ref[...] = lax.dot_general(
        x_ref[...], w_ref[...], (((1,), (0,)), ((), ())),
        preferred_element_type=jnp.float32)


def _mm1(x, W1):
    return pl.pallas_call(
        _mm1_body,
        grid=(5,),
        in_specs=[pl.BlockSpec((N_PAD // 5, F), lambda i: (i, 0)),
                  pl.BlockSpec((F, H1), lambda i: (0, 0))],
        out_specs=pl.BlockSpec((N_PAD // 5, H1), lambda i: (i, 0)),
        out_shape=jax.ShapeDtypeStruct((N_PAD, H1), jnp.float32),
    )(x, W1)


def _comb2_body(p0_ref, p1_ref, w_ref, o_ref):
    h = jnp.maximum(p0_ref[...] + p1_ref[...], 0.0)
    o_ref[...] = lax.dot_general(
        h, w_ref[...], (((1,), (0,)), ((), ())),
        preferred_element_type=jnp.float32)


def _comb2(p0, p1, W23):
    return pl.pallas_call(
        _comb2_body,
        grid=(5,),
        in_specs=[pl.BlockSpec((N_PAD // 5, H1), lambda i: (i, 0)),
                  pl.BlockSpec((N_PAD // 5, H1), lambda i: (i, 0)),
                  pl.BlockSpec((H1, H1), lambda i: (0, 0))],
        out_specs=pl.BlockSpec((N_PAD // 5, H1), lambda i: (i, 0)),
        out_shape=jax.ShapeDtypeStruct((N_PAD, H1), jnp.float32),
    )(p0, p1, W23)


def _reparam_body(q0_ref, q1_ref, eps_ref, o_ref):
    s = q0_ref[...] + q1_ref[...]
    o_ref[...] = s[:, :H2] + eps_ref[...] * jnp.exp(s[:, H2:])


def _reparam(q0, q1, eps):
    return pl.pallas_call(
        _reparam_body,
        grid=(5,),
        in_specs=[pl.BlockSpec((2000, H1), lambda i: (i, 0)),
                  pl.BlockSpec((2000, H1), lambda i: (i, 0)),
                  pl.BlockSpec((2000, H2), lambda i: (i, 0))],
        out_specs=pl.BlockSpec((2000, H2), lambda i: (i, 0)),
        out_shape=jax.ShapeDtypeStruct((N, H2), jnp.float32),
    )(q0, q1, eps)


_ZB = 128
_GB = (N + _ZB - 1) // _ZB
_NP = _GB * _ZB
_OBR = _ZB * N // 128


def _decode_body(zb_ref, ztr_ref, out_ref):
    lane = lax.broadcasted_iota(jnp.int32, (16, 128), 1)
    d = [lax.dot_general(zb_ref[16 * s:16 * s + 16, :], ztr_ref[s],
                         (((1,), (0,)), ((), ())),
                         preferred_element_type=jnp.float32)
         for s in range(8)]
    for s in range(8):
        main = jnp.reshape(d[s][:, :9984], (1248, 128))
        for m in range(16):
            out_ref[pl.ds(625 * m + 78 * s, 78), :] = main[78 * m:78 * m + 78, :]
    for s in range(8):
        tail = d[s][:, 9984:10112]
        if s < 7:
            bnd = jnp.where(lane < 16 * (s + 1), tail, d[s + 1][:, 0:128])
        else:
            bnd = tail
        for m in range(16):
            out_ref[pl.ds(625 * m + 78 * s + 78, 1), :] = bnd[m:m + 1, :]


def _decode(z):
    zp = jnp.concatenate([z, jnp.zeros((_NP - N, H2), jnp.float32)], axis=0)
    zperm = jnp.reshape(
        jnp.transpose(jnp.reshape(zp, (_GB, 16, 8, H2)), (0, 2, 1, 3)),
        (_NP, H2))
    zT = z.T
    rolls = jnp.stack([jnp.roll(zT, 16 * s, axis=1) for s in range(8)])
    ztr = jnp.concatenate([rolls, rolls[:, :, :112]], axis=2)
    out2 = pl.pallas_call(
        _decode_body,
        grid=(_GB,),
        in_specs=[pl.BlockSpec((_ZB, H2), lambda g: (g, 0)),
                  pl.BlockSpec((8, H2, N + 112), lambda g: (0, 0, 0))],
        out_specs=pl.BlockSpec((_OBR, 128), lambda g: (g, 0)),
        out_shape=jax.ShapeDtypeStruct((N * N // 128, 128), jnp.float32),
    )(zperm, ztr)
    return jnp.reshape(out2, (N * N,))


def kernel(x, edge_index, edge_weight, W1, W2, W3):
    src = edge_index[1]
    dst = edge_index[0]
    pad = E_PAD - E
    src_p = jnp.concatenate(
        [src, jnp.zeros((pad,), jnp.int32)]).reshape(NW, CPT, CHUNK)
    dst_p = jnp.concatenate(
        [dst, jnp.zeros((pad,), jnp.int32)]).reshape(NW, CPT, CHUNK)
    ew_p = jnp.concatenate(
        [edge_weight, jnp.zeros((pad,), jnp.float32)])
    zeros_nh = jnp.zeros((N_PAD, H1), jnp.float32)
    W23 = jnp.concatenate([W2, W3], axis=1)
    eps = jax.random.normal(jax.random.key(42), (N, H2), dtype=jnp.float32)

    x_pad = jnp.concatenate(
        [x, jnp.zeros((N_PAD - N, F), jnp.float32)], axis=0)

    hw1 = _mm1(x_pad, W1)
    p = _sc_agg(hw1, src_p, dst_p, ew_p, zeros_nh)
    hw23 = _comb2(p[0], p[1], W23)
    q = _sc_agg(hw23, src_p, dst_p, ew_p, zeros_nh)
    z = _reparam(q[0, :N], q[1, :N], eps)
    return _decode(z)
